# baseline (speedup 1.0000x reference)
import jax
import jax.numpy as jnp
from jax.experimental import pallas as pl
from jax.experimental.pallas import tpu as pltpu

_TM = 200


def _body(x_ref, w1_ref, b1_ref, w2_ref, b2_ref, adj_ref, out_ref,
          fts_ref, h1_ref):
    layer = pl.program_id(0)
    i = pl.program_id(1)

    @pl.when((layer == 0) & (i == 0))
    def _():
        fts_ref[...] = (
            jnp.dot(x_ref[...].astype(jnp.bfloat16),
                    w1_ref[...].astype(jnp.bfloat16),
                    preferred_element_type=jnp.float32)
            + b1_ref[...]
        ).astype(jnp.bfloat16)

    @pl.when((layer == 1) & (i == 0))
    def _():
        fts_ref[...] = (
            jnp.dot(h1_ref[...], w2_ref[...].astype(jnp.bfloat16),
                    preferred_element_type=jnp.float32)
            + b2_ref[...]
        ).astype(jnp.bfloat16)

    act = jnp.maximum(
        jnp.dot(adj_ref[...].astype(jnp.bfloat16), fts_ref[...],
                preferred_element_type=jnp.float32),
        0.0,
    )

    @pl.when(layer == 0)
    def _():
        h1_ref[pl.ds(i * _TM, _TM), :] = act.astype(jnp.bfloat16)

    out_ref[...] = act


def kernel(seq, adj, W1, b1, W2, b2):
    x = jnp.squeeze(seq, axis=0)
    n = adj.shape[0]
    d = W1.shape[1]
    out = pl.pallas_call(
        _body,
        grid=(2, n // _TM),
        in_specs=[
            pl.BlockSpec((n, W1.shape[0]), lambda l, i: (0, 0)),
            pl.BlockSpec(W1.shape, lambda l, i: (0, 0)),
            pl.BlockSpec((1, d), lambda l, i: (0, 0)),
            pl.BlockSpec(W2.shape, lambda l, i: (0, 0)),
            pl.BlockSpec((1, d), lambda l, i: (0, 0)),
            pl.BlockSpec((_TM, n), lambda l, i: (i, 0)),
        ],
        out_specs=pl.BlockSpec((_TM, d), lambda l, i: (i, 0)),
        out_shape=jax.ShapeDtypeStruct((n, d), jnp.float32),
        scratch_shapes=[
            pltpu.VMEM((n, d), jnp.bfloat16),
            pltpu.VMEM((n, d), jnp.bfloat16),
        ],
    )(x, W1, b1.reshape(1, -1), W2, b2.reshape(1, -1), adj)
    return out[None, :, :]


# speedup vs baseline: 1.0048x; 1.0048x over previous
import jax
import jax.numpy as jnp
from jax.experimental import pallas as pl
from jax.experimental.pallas import tpu as pltpu

_TM = 400


def _body(x_ref, w1_ref, b1_ref, w2_ref, b2_ref, adj_ref, out_ref,
          fts1_ref, fts2_ref):
    layer = pl.program_id(0)
    i = pl.program_id(1)

    @pl.when((layer == 0) & (i == 0))
    def _():
        fts1_ref[...] = (
            jnp.dot(x_ref[...].astype(jnp.bfloat16),
                    w1_ref[...].astype(jnp.bfloat16),
                    preferred_element_type=jnp.float32)
            + b1_ref[...]
        ).astype(jnp.bfloat16)

    adj_bf16 = adj_ref[...].astype(jnp.bfloat16)

    @pl.when(layer == 0)
    def _():
        act = jnp.maximum(
            jnp.dot(adj_bf16, fts1_ref[...], preferred_element_type=jnp.float32),
            0.0,
        )
        out_ref[...] = act
        fts2_ref[pl.ds(i * _TM, _TM), :] = (
            jnp.dot(act.astype(jnp.bfloat16), w2_ref[...].astype(jnp.bfloat16),
                    preferred_element_type=jnp.float32)
            + b2_ref[...]
        ).astype(jnp.bfloat16)

    @pl.when(layer == 1)
    def _():
        out_ref[...] = jnp.maximum(
            jnp.dot(adj_bf16, fts2_ref[...], preferred_element_type=jnp.float32),
            0.0,
        )


def kernel(seq, adj, W1, b1, W2, b2):
    x = jnp.squeeze(seq, axis=0)
    n = adj.shape[0]
    d = W1.shape[1]
    out = pl.pallas_call(
        _body,
        grid=(2, n // _TM),
        in_specs=[
            pl.BlockSpec((n, W1.shape[0]), lambda l, i: (0, 0)),
            pl.BlockSpec(W1.shape, lambda l, i: (0, 0)),
            pl.BlockSpec((1, d), lambda l, i: (0, 0)),
            pl.BlockSpec(W2.shape, lambda l, i: (0, 0)),
            pl.BlockSpec((1, d), lambda l, i: (0, 0)),
            pl.BlockSpec((_TM, n), lambda l, i: (i, 0)),
        ],
        out_specs=pl.BlockSpec((_TM, d), lambda l, i: (i, 0)),
        out_shape=jax.ShapeDtypeStruct((n, d), jnp.float32),
        scratch_shapes=[
            pltpu.VMEM((n, d), jnp.bfloat16),
            pltpu.VMEM((n, d), jnp.bfloat16),
        ],
    )(x, W1, b1.reshape(1, -1), W2, b2.reshape(1, -1), adj)
    return out[None, :, :]


# speedup vs baseline: 1.0249x; 1.0200x over previous
import jax
import jax.numpy as jnp
from jax.experimental import pallas as pl
from jax.experimental.pallas import tpu as pltpu

_TM = 400


def _body(x_ref, w1_ref, b1_ref, w2_ref, b2_ref, adj_ref, out_ref,
          fts_ref, h1_ref):
    layer = pl.program_id(0)
    i = pl.program_id(1)

    @pl.when((layer == 0) & (i == 0))
    def _():
        fts_ref[...] = (
            jnp.dot(x_ref[...], w1_ref[...],
                    preferred_element_type=jnp.float32)
            + b1_ref[...]
        )

    @pl.when((layer == 1) & (i == 0))
    def _():
        fts_ref[...] = (
            jnp.dot(h1_ref[...], w2_ref[...],
                    preferred_element_type=jnp.float32)
            + b2_ref[...]
        )

    act = jnp.maximum(
        jnp.dot(adj_ref[...], fts_ref[...],
                preferred_element_type=jnp.float32),
        0.0,
    )

    @pl.when(layer == 0)
    def _():
        h1_ref[pl.ds(i * _TM, _TM), :] = act

    out_ref[...] = act


def kernel(seq, adj, W1, b1, W2, b2):
    x = jnp.squeeze(seq, axis=0)
    n = adj.shape[0]
    d = W1.shape[1]
    out = pl.pallas_call(
        _body,
        grid=(2, n // _TM),
        in_specs=[
            pl.BlockSpec((n, W1.shape[0]), lambda l, i: (0, 0)),
            pl.BlockSpec(W1.shape, lambda l, i: (0, 0)),
            pl.BlockSpec((1, d), lambda l, i: (0, 0)),
            pl.BlockSpec(W2.shape, lambda l, i: (0, 0)),
            pl.BlockSpec((1, d), lambda l, i: (0, 0)),
            pl.BlockSpec((_TM, n), lambda l, i: (i, 0)),
        ],
        out_specs=pl.BlockSpec((_TM, d), lambda l, i: (i, 0)),
        out_shape=jax.ShapeDtypeStruct((n, d), jnp.float32),
        scratch_shapes=[
            pltpu.VMEM((n, d), jnp.float32),
            pltpu.VMEM((n, d), jnp.float32),
        ],
    )(x, W1, b1.reshape(1, -1), W2, b2.reshape(1, -1), adj)
    return out[None, :, :]


# speedup vs baseline: 1.0315x; 1.0065x over previous
import jax
import jax.numpy as jnp
from jax.experimental import pallas as pl
from jax.experimental.pallas import tpu as pltpu

_TM = 400


def _body(x_ref, w1_ref, b1_ref, w2_ref, b2_ref, adj_ref, out_ref,
          fts1_ref, h1_ref, fts2_ref):
    layer = pl.program_id(0)
    i = pl.program_id(1)

    @pl.when((layer == 0) & (i == 0))
    def _():
        fts1_ref[...] = (
            jnp.dot(x_ref[...], w1_ref[...],
                    preferred_element_type=jnp.float32)
            + b1_ref[...]
        )

    @pl.when(layer == 0)
    def _():
        act = jnp.maximum(
            jnp.dot(adj_ref[...], fts1_ref[...],
                    preferred_element_type=jnp.float32),
            0.0,
        )
        h1_ref[pl.ds(i * _TM, _TM), :] = act
        out_ref[...] = act

    @pl.when((layer == 0) & (i >= 17) & (i < 24))
    def _():
        j = i - 17
        fts2_ref[pl.ds(j * 1200, 1200), :] = (
            jnp.dot(h1_ref[pl.ds(j * 1200, 1200), :], w2_ref[...],
                    preferred_element_type=jnp.float32)
            + b2_ref[...]
        )

    @pl.when((layer == 0) & (i == 24))
    def _():
        fts2_ref[pl.ds(8400, 1600), :] = (
            jnp.dot(h1_ref[pl.ds(8400, 1600), :], w2_ref[...],
                    preferred_element_type=jnp.float32)
            + b2_ref[...]
        )

    @pl.when(layer == 1)
    def _():
        out_ref[...] = jnp.maximum(
            jnp.dot(adj_ref[...], fts2_ref[...],
                    preferred_element_type=jnp.float32),
            0.0,
        )


def kernel(seq, adj, W1, b1, W2, b2):
    x = jnp.squeeze(seq, axis=0)
    n = adj.shape[0]
    d = W1.shape[1]
    out = pl.pallas_call(
        _body,
        grid=(2, n // _TM),
        in_specs=[
            pl.BlockSpec((n, W1.shape[0]), lambda l, i: (0, 0)),
            pl.BlockSpec(W1.shape, lambda l, i: (0, 0)),
            pl.BlockSpec((1, d), lambda l, i: (0, 0)),
            pl.BlockSpec(W2.shape, lambda l, i: (0, 0)),
            pl.BlockSpec((1, d), lambda l, i: (0, 0)),
            pl.BlockSpec((_TM, n), lambda l, i: (i, 0)),
        ],
        out_specs=pl.BlockSpec((_TM, d), lambda l, i: (i * l, 0)),
        out_shape=jax.ShapeDtypeStruct((n, d), jnp.float32),
        scratch_shapes=[
            pltpu.VMEM((n, d), jnp.float32),
            pltpu.VMEM((n, d), jnp.float32),
            pltpu.VMEM((n, d), jnp.float32),
        ],
    )(x, W1, b1.reshape(1, -1), W2, b2.reshape(1, -1), adj)
    return out[None, :, :]
